# Initial kernel scaffold; baseline (speedup 1.0000x reference)
import jax
import jax.numpy as jnp
from jax.experimental import pallas as pl


def kernel(x, edge_index, W1, b1, W2, b2):
    raise NotImplementedError("write your pallas kernel here")


# speedup vs baseline: 23.6988x; 23.6988x over previous
import functools

import jax
import jax.numpy as jnp
from jax import lax
from jax.experimental import pallas as pl
from jax.experimental.pallas import tpu as pltpu
from jax.experimental.pallas import tpu_sc as plsc

N = 10000
E = 320000
DI = 128
DH = 128
DO = 64

NP = 10240
NC = 2
NS = 16
NW = NC * NS
B = 128
NB = 80
EP = NW * NB * B
RPS = NP // NS

@functools.lru_cache(maxsize=None)
def _mesh():
    return plsc.VectorSubcoreMesh(
        core_axis_name="c", subcore_axis_name="s", num_cores=NC, num_subcores=NS
    )


def _deg_body(init_hbm, dst_hbm, out_hbm, dst_v, ones_v, acc, sem):
    c = lax.axis_index("c")
    s = lax.axis_index("s")
    wid = s * NC + c
    pltpu.sync_copy(dst_hbm.at[wid], dst_v)
    for t in range(B // 16):
        ones_v[pl.ds(t * 16, 16)] = jnp.ones((16,), jnp.float32)
    pltpu.sync_copy(init_hbm.at[c, pl.ds(s * RPS, RPS)], acc.at[pl.ds(s * RPS, RPS)])
    plsc.subcore_barrier()

    def body(j, carry):
        pltpu.sync_copy(ones_v, acc.at[dst_v.at[j]], add=True)
        return carry

    lax.fori_loop(0, NB, body, 0)
    plsc.subcore_barrier()
    pltpu.sync_copy(acc.at[pl.ds(s * RPS, RPS)], out_hbm.at[c, pl.ds(s * RPS, RPS)])


@functools.lru_cache(maxsize=None)
def _deg_kernel():
    return pl.kernel(
        _deg_body,
        out_type=jax.ShapeDtypeStruct((NC, NP), jnp.float32),
        mesh=_mesh(),
        scratch_types=[
            pltpu.VMEM((NB, B), jnp.int32),
            pltpu.VMEM((B,), jnp.float32),
            pltpu.VMEM_SHARED((NP,), jnp.float32),
            pltpu.SemaphoreType.DMA,
        ],
    )


def _agg_body(ys_hbm, zero_hbm, src_hbm, dst_hbm, out_hbm,
              src_v, dst_v, rows_v, acc, sem):
    c = lax.axis_index("c")
    s = lax.axis_index("s")
    wid = s * NC + c

    pltpu.sync_copy(src_hbm.at[wid], src_v)
    pltpu.sync_copy(dst_hbm.at[wid], dst_v)

    @pl.when(c == 0)
    def _():
        pltpu.sync_copy(ys_hbm.at[pl.ds(s * RPS, RPS)], acc.at[pl.ds(s * RPS, RPS)])

    @pl.when(c == 1)
    def _():
        pltpu.sync_copy(zero_hbm.at[pl.ds(s * RPS, RPS)], acc.at[pl.ds(s * RPS, RPS)])

    plsc.subcore_barrier()

    def body(j, carry):
        pltpu.async_copy(ys_hbm.at[src_v.at[j]], rows_v, sem).wait()
        pltpu.sync_copy(rows_v, acc.at[dst_v.at[j]], add=True)
        return carry

    lax.fori_loop(0, NB, body, 0)
    plsc.subcore_barrier()
    pltpu.sync_copy(acc.at[pl.ds(s * RPS, RPS)], out_hbm.at[c, pl.ds(s * RPS, RPS)])


@functools.lru_cache(maxsize=None)
def _make_agg(d):
    return pl.kernel(
        _agg_body,
        out_type=jax.ShapeDtypeStruct((NC, NP, d), jnp.float32),
        mesh=_mesh(),
        scratch_types=[
            pltpu.VMEM((NB, B), jnp.int32),
            pltpu.VMEM((NB, B), jnp.int32),
            pltpu.VMEM((B, d), jnp.float32),
            pltpu.VMEM_SHARED((NP, d), jnp.float32),
            pltpu.SemaphoreType.DMA,
        ],
        compiler_params=pltpu.CompilerParams(use_tc_tiling_on_sc=False),
    )


def _tc_pre_body(x_ref, w_ref, d0_ref, d1_ref, ys_ref, dis_ref):
    deg = d0_ref[...] + d1_ref[...]
    dis = jnp.where(deg > 0.0, lax.rsqrt(deg), 0.0)
    xw = jnp.dot(x_ref[...], w_ref[...], preferred_element_type=jnp.float32)
    ys_ref[...] = xw * dis
    dis_ref[...] = dis


def _tc_mid_body(p0_ref, p1_ref, dis_ref, b1_ref, w_ref, ys2_ref):
    dis = dis_ref[...]
    h = jnp.maximum(dis * (p0_ref[...] + p1_ref[...]) + b1_ref[...], 0.0)
    hw = jnp.dot(h, w_ref[...], preferred_element_type=jnp.float32)
    ys2_ref[...] = hw * dis


def _tc_post_body(q0_ref, q1_ref, dis_ref, b2_ref, out_ref):
    dis = dis_ref[...]
    out_ref[...] = dis * (q0_ref[...] + q1_ref[...]) + b2_ref[...]


def _tc_pre(x_pad, W1, d0, d1):
    R = 1024
    return pl.pallas_call(
        _tc_pre_body,
        grid=(NP // R,),
        in_specs=[
            pl.BlockSpec((R, DI), lambda i: (i, 0)),
            pl.BlockSpec((DI, DH), lambda i: (0, 0)),
            pl.BlockSpec((R, 1), lambda i: (i, 0)),
            pl.BlockSpec((R, 1), lambda i: (i, 0)),
        ],
        out_specs=[
            pl.BlockSpec((R, DH), lambda i: (i, 0)),
            pl.BlockSpec((R, 1), lambda i: (i, 0)),
        ],
        out_shape=[
            jax.ShapeDtypeStruct((NP, DH), jnp.float32),
            jax.ShapeDtypeStruct((NP, 1), jnp.float32),
        ],
    )(x_pad, W1, d0, d1)


def _tc_mid(p0, p1, dis, b1, W2):
    R = 1024
    return pl.pallas_call(
        _tc_mid_body,
        grid=(NP // R,),
        in_specs=[
            pl.BlockSpec((R, DH), lambda i: (i, 0)),
            pl.BlockSpec((R, DH), lambda i: (i, 0)),
            pl.BlockSpec((R, 1), lambda i: (i, 0)),
            pl.BlockSpec((1, DH), lambda i: (0, 0)),
            pl.BlockSpec((DH, DO), lambda i: (0, 0)),
        ],
        out_specs=pl.BlockSpec((R, DO), lambda i: (i, 0)),
        out_shape=jax.ShapeDtypeStruct((NP, DO), jnp.float32),
    )(p0, p1, dis, b1, W2)


def _tc_post(q0, q1, dis, b2):
    R = 1000
    return pl.pallas_call(
        _tc_post_body,
        grid=(N // R,),
        in_specs=[
            pl.BlockSpec((R, DO), lambda i: (i, 0)),
            pl.BlockSpec((R, DO), lambda i: (i, 0)),
            pl.BlockSpec((R, 1), lambda i: (i, 0)),
            pl.BlockSpec((1, DO), lambda i: (0, 0)),
        ],
        out_specs=pl.BlockSpec((R, DO), lambda i: (i, 0)),
        out_shape=jax.ShapeDtypeStruct((N, DO), jnp.float32),
    )(q0, q1, dis, b2)


def kernel(x, edge_index, W1, b1, W2, b2):
    src = edge_index[0].astype(jnp.int32)
    dst = edge_index[1].astype(jnp.int32)
    pad_i = jnp.arange(EP - E, dtype=jnp.int32)
    src_p = jnp.concatenate([src, (pad_i * 97) % N])
    dst_p = jnp.concatenate([dst, N + pad_i % (NP - N)])
    src_r = src_p.reshape(NW, NB, B)
    dst_r = dst_p.reshape(NW, NB, B)

    x_pad = jnp.pad(x, ((0, NP - N), (0, 0)))
    deg_init = jnp.concatenate(
        [jnp.ones((1, NP), jnp.float32), jnp.zeros((1, NP), jnp.float32)]
    )

    degp = _deg_kernel()(deg_init, dst_r)
    ys, dis = _tc_pre(x_pad, W1,
                      degp[0].reshape(NP, 1), degp[1].reshape(NP, 1))
    p = _make_agg(DH)(ys, jnp.zeros((NP, DH), jnp.float32), src_r, dst_r)
    ys2 = _tc_mid(p[0], p[1], dis, b1.reshape(1, DH), W2)
    q = _make_agg(DO)(ys2, jnp.zeros((NP, DO), jnp.float32), src_r, dst_r)
    out = _tc_post(q[0], q[1], dis, b2.reshape(1, DO))
    return out


# speedup vs baseline: 28.7389x; 1.2127x over previous
import functools

import jax
import jax.numpy as jnp
from jax import lax
from jax.experimental import pallas as pl
from jax.experimental.pallas import tpu as pltpu
from jax.experimental.pallas import tpu_sc as plsc

N = 10000
E = 320000
DI = 128
DH = 128
DO = 64

NP = 10240
NC = 2
NS = 16
NW = NC * NS
B = 128
NB = 80
NB2 = 160
EP = NW * NB * B
RPS = NP // NS


@functools.lru_cache(maxsize=None)
def _mesh():
    return plsc.VectorSubcoreMesh(
        core_axis_name="c", subcore_axis_name="s", num_cores=NC, num_subcores=NS
    )


def _deg_body(init_hbm, dst_hbm, out_hbm, dst_v, ones_v, acc, sem):
    c = lax.axis_index("c")
    s = lax.axis_index("s")
    wid = s * NC + c
    pltpu.sync_copy(dst_hbm.at[wid], dst_v)
    for t in range(B // 16):
        ones_v[pl.ds(t * 16, 16)] = jnp.ones((16,), jnp.float32)
    pltpu.sync_copy(init_hbm.at[c, pl.ds(s * RPS, RPS)], acc.at[pl.ds(s * RPS, RPS)])
    plsc.subcore_barrier()

    def body(j, carry):
        pltpu.sync_copy(ones_v, acc.at[dst_v.at[j]], add=True)
        return carry

    lax.fori_loop(0, NB, body, 0)
    plsc.subcore_barrier()
    pltpu.sync_copy(acc.at[pl.ds(s * RPS, RPS)], out_hbm.at[c, pl.ds(s * RPS, RPS)])


@functools.lru_cache(maxsize=None)
def _deg_kernel():
    return pl.kernel(
        _deg_body,
        out_type=jax.ShapeDtypeStruct((NC, NP), jnp.float32),
        mesh=_mesh(),
        scratch_types=[
            pltpu.VMEM((NB, B), jnp.int32),
            pltpu.VMEM((B,), jnp.float32),
            pltpu.VMEM_SHARED((NP,), jnp.float32),
            pltpu.SemaphoreType.DMA,
        ],
        compiler_params=pltpu.CompilerParams(use_tc_tiling_on_sc=False),
    )


def _agg_body(ys_hbm, src0_hbm, src1_hbm, dst_hbm, out_hbm,
              src_v, dst_v, rows0_v, rows1_v, acc, sem0, sem1):
    c = lax.axis_index("c")
    s = lax.axis_index("s")

    @pl.when(c == 0)
    def _():
        pltpu.sync_copy(src0_hbm.at[s], src_v)

    @pl.when(c == 1)
    def _():
        pltpu.sync_copy(src1_hbm.at[s], src_v)

    pltpu.sync_copy(dst_hbm.at[s], dst_v)

    pltpu.sync_copy(ys_hbm.at[pl.ds(c * NP + s * RPS, RPS)],
                    acc.at[pl.ds(s * RPS, RPS)])
    plsc.subcore_barrier()

    pltpu.async_copy(ys_hbm.at[src_v.at[0]], rows0_v, sem0)

    def body(jj, carry):
        j = jj * 2
        d1 = pltpu.async_copy(ys_hbm.at[src_v.at[j + 1]], rows1_v, sem1)
        pltpu.make_async_copy(ys_hbm.at[src_v.at[j]], rows0_v, sem0).wait()
        pltpu.sync_copy(rows0_v, acc.at[dst_v.at[j]], add=True)

        @pl.when(jj + 1 < NB2 // 2)
        def _():
            pltpu.async_copy(ys_hbm.at[src_v.at[j + 2]], rows0_v, sem0)

        d1.wait()
        pltpu.sync_copy(rows1_v, acc.at[dst_v.at[j + 1]], add=True)
        return carry

    lax.fori_loop(0, NB2 // 2, body, 0)
    plsc.subcore_barrier()
    pltpu.sync_copy(acc.at[pl.ds(s * RPS, RPS)], out_hbm.at[c, pl.ds(s * RPS, RPS)])


@functools.lru_cache(maxsize=None)
def _make_agg(d):
    return pl.kernel(
        _agg_body,
        out_type=jax.ShapeDtypeStruct((NC, NP, d), jnp.float32),
        mesh=_mesh(),
        scratch_types=[
            pltpu.VMEM((NB2, B), jnp.int32),
            pltpu.VMEM((NB2, B), jnp.int32),
            pltpu.VMEM((B, d), jnp.float32),
            pltpu.VMEM((B, d), jnp.float32),
            pltpu.VMEM_SHARED((NP, d), jnp.float32),
            pltpu.SemaphoreType.DMA,
            pltpu.SemaphoreType.DMA,
        ],
        compiler_params=pltpu.CompilerParams(use_tc_tiling_on_sc=False),
    )


def _tc_pre_body(x_ref, w_ref, d0_ref, d1_ref, ys_ref, dis_ref):
    deg = d0_ref[...] + d1_ref[...]
    dis = jnp.where(deg > 0.0, lax.rsqrt(deg), 0.0)
    xw = jnp.dot(x_ref[...], w_ref[...], preferred_element_type=jnp.float32)
    ys = xw * dis
    h = DH // 2
    ys_ref[...] = jnp.stack([ys[:, :h], ys[:, h:]])
    dis_ref[...] = dis


def _tc_mid_body(p_ref, dis_ref, b1_ref, w_ref, ys2_ref):
    dis = dis_ref[...]
    agg = jnp.concatenate([p_ref[0], p_ref[1]], axis=1)
    h = jnp.maximum(dis * agg + b1_ref[...], 0.0)
    hw = jnp.dot(h, w_ref[...], preferred_element_type=jnp.float32)
    ys2 = hw * dis
    ho = DO // 2
    ys2_ref[...] = jnp.stack([ys2[:, :ho], ys2[:, ho:]])


def _tc_post_body(q_ref, dis_ref, b2_ref, out_ref):
    agg = jnp.concatenate([q_ref[0], q_ref[1]], axis=1)
    out_ref[...] = dis_ref[...] * agg + b2_ref[...]


def _tc_pre(x_pad, W1, d0, d1):
    R = 1024
    return pl.pallas_call(
        _tc_pre_body,
        grid=(NP // R,),
        in_specs=[
            pl.BlockSpec((R, DI), lambda i: (i, 0)),
            pl.BlockSpec((DI, DH), lambda i: (0, 0)),
            pl.BlockSpec((R, 1), lambda i: (i, 0)),
            pl.BlockSpec((R, 1), lambda i: (i, 0)),
        ],
        out_specs=[
            pl.BlockSpec((NC, R, DH // 2), lambda i: (0, i, 0)),
            pl.BlockSpec((R, 1), lambda i: (i, 0)),
        ],
        out_shape=[
            jax.ShapeDtypeStruct((NC, NP, DH // 2), jnp.float32),
            jax.ShapeDtypeStruct((NP, 1), jnp.float32),
        ],
    )(x_pad, W1, d0, d1)


def _tc_mid(p, dis, b1, W2):
    R = 1024
    return pl.pallas_call(
        _tc_mid_body,
        grid=(NP // R,),
        in_specs=[
            pl.BlockSpec((NC, R, DH // 2), lambda i: (0, i, 0)),
            pl.BlockSpec((R, 1), lambda i: (i, 0)),
            pl.BlockSpec((1, DH), lambda i: (0, 0)),
            pl.BlockSpec((DH, DO), lambda i: (0, 0)),
        ],
        out_specs=pl.BlockSpec((NC, R, DO // 2), lambda i: (0, i, 0)),
        out_shape=jax.ShapeDtypeStruct((NC, NP, DO // 2), jnp.float32),
    )(p, dis, b1, W2)


def _tc_post(q, dis, b2):
    R = 1000
    return pl.pallas_call(
        _tc_post_body,
        grid=(N // R,),
        in_specs=[
            pl.BlockSpec((NC, R, DO // 2), lambda i: (0, i, 0)),
            pl.BlockSpec((R, 1), lambda i: (i, 0)),
            pl.BlockSpec((1, DO), lambda i: (0, 0)),
        ],
        out_specs=pl.BlockSpec((R, DO), lambda i: (i, 0)),
        out_shape=jax.ShapeDtypeStruct((N, DO), jnp.float32),
    )(q, dis, b2)


def kernel(x, edge_index, W1, b1, W2, b2):
    src = edge_index[0].astype(jnp.int32)
    dst = edge_index[1].astype(jnp.int32)
    pad_i = jnp.arange(EP - E, dtype=jnp.int32)
    src_p = jnp.concatenate([src, (pad_i * 97) % N])
    dst_p = jnp.concatenate([dst, N + pad_i % (NP - N)])
    dst_deg = dst_p.reshape(NW, NB, B)
    src_r0 = src_p.reshape(NS, NB2, B)
    src_r1 = src_r0 + NP
    dst_r = dst_p.reshape(NS, NB2, B)

    x_pad = jnp.pad(x, ((0, NP - N), (0, 0)))
    deg_init = jnp.concatenate(
        [jnp.ones((1, NP), jnp.float32), jnp.zeros((1, NP), jnp.float32)]
    )

    degp = _deg_kernel()(deg_init, dst_deg)
    ys, dis = _tc_pre(x_pad, W1,
                      degp[0].reshape(NP, 1), degp[1].reshape(NP, 1))
    ys_cat = ys.reshape(NC * NP, DH // 2)
    p = _make_agg(DH // 2)(ys_cat, src_r0, src_r1, dst_r)
    ys2 = _tc_mid(p, dis, b1.reshape(1, DH), W2)
    ys2_cat = ys2.reshape(NC * NP, DO // 2)
    q = _make_agg(DO // 2)(ys2_cat, src_r0, src_r1, dst_r)
    out = _tc_post(q, dis, b2.reshape(1, DO))
    return out
